# baseline (speedup 1.0000x reference)
import jax
import jax.numpy as jnp
from jax.experimental import pallas as pl


_BT = 1024


def _add_pe_kernel(x_ref, pe_ref, o_ref):
    o_ref[...] = x_ref[...] + pe_ref[...][None, :, :]


def kernel(x, pe_table):
    B, T, C = x.shape
    grid = (T // _BT,)
    return pl.pallas_call(
        _add_pe_kernel,
        grid=grid,
        in_specs=[
            pl.BlockSpec((4, _BT, C), lambda t: (0, t, 0)),
            pl.BlockSpec((_BT, C), lambda t: (t, 0)),
        ],
        out_specs=pl.BlockSpec((4, _BT, C), lambda t: (0, t, 0)),
        out_shape=jax.ShapeDtypeStruct((B, T, C), x.dtype),
    )(x, pe_table[:T])


# speedup vs baseline: 1.0026x; 1.0026x over previous
import jax
from jax.experimental import pallas as pl


_BT = 1024


def _add_pe_kernel(x_ref, pe_ref, o_ref):
    o_ref[...] = x_ref[...] + pe_ref[...][None, :, :]


def kernel(x, pe_table):
    B, T, C = x.shape
    grid = (T // _BT,)
    return pl.pallas_call(
        _add_pe_kernel,
        grid=grid,
        in_specs=[
            pl.BlockSpec((B, _BT, C), lambda t: (0, t, 0)),
            pl.BlockSpec((_BT, C), lambda t: (t, 0)),
        ],
        out_specs=pl.BlockSpec((B, _BT, C), lambda t: (0, t, 0)),
        out_shape=jax.ShapeDtypeStruct((B, T, C), x.dtype),
    )(x, pe_table[:T])
